# baseline (speedup 1.0000x reference)
import functools

import jax
import jax.numpy as jnp
from jax import lax
from jax.experimental import pallas as pl
from jax.experimental.pallas import tpu as pltpu
from jax.experimental.pallas import tpu_sc as plsc

_L = 16
_CHUNK = 128


def _sc_body(nc, ns, bpw, n_rows, r_rules,
             x_hbm, mat_hbm, rp_hbm, wb_hbm, out_hbm, slk_hbm,
             xv, matv2, pfidx, pv, resv, wbv, riota, slkv, sem, sem_p):
    wid = lax.axis_index("s") * nc + lax.axis_index("c")
    base = wid * bpw
    n_groups = bpw // _L
    n_chunks_m = bpw // _CHUNK
    n_chunks = (bpw * r_rules) // _CHUNK
    gpc = _CHUNK // _L
    chunks_per_rule = n_chunks // r_rules

    lanes = lax.iota(jnp.int32, _L)

    pltpu.sync_copy(x_hbm.at[pl.ds(base, bpw)], xv)
    pltpu.sync_copy(wb_hbm, wbv)

    def fill_r(r9, _):
        off = pl.multiple_of(r9 * _L, _L)
        riota[pl.ds(off, _L)] = jnp.zeros((_L,), jnp.int32) + r9
        return 0

    lax.fori_loop(0, r_rules, fill_r, 0)

    m_descs = [
        pltpu.async_copy(
            mat_hbm.at[xv.at[pl.ds(mm * _CHUNK, _CHUNK)]],
            matv2.at[pl.ds(mm * _CHUNK, _CHUNK), :],
            sem,
        )
        for mm in range(n_chunks_m)
    ]

    def build_pfidx(k, _):
        r = k // n_groups
        j = k - r * n_groups
        rows = j * _L + lanes
        cols = riota[pl.ds(pl.multiple_of(r * _L, _L), _L)]
        m = plsc.load_gather(matv2, [rows, cols])
        pidx = jnp.where(m == m, m, 0.0).astype(jnp.int32)
        row = k // gpc
        col = pl.multiple_of((k - row * gpc) * _L, _L)
        pfidx[row, pl.ds(col, _L)] = pidx
        return 0

    p_descs = []
    for mm in range(n_chunks_m):
        m_descs[mm].wait()
        for r in range(r_rules):
            cc = r * chunks_per_rule + mm
            lax.fori_loop(cc * gpc, (cc + 1) * gpc, build_pfidx, 0)
            p_descs.append(pltpu.async_copy(
                rp_hbm.at[r].at[pfidx.at[cc]],
                pv.at[pl.ds(cc * _CHUNK, _CHUNK)],
                sem_p,
            ))
    for d in p_descs:
        d.wait()

    wv_all = wbv[...]

    def bcast(i):
        s = jnp.sum(jnp.where(lanes == i, wv_all, 0.0))
        return jnp.zeros((_L,), jnp.float32) + s

    w_vecs = [bcast(r) for r in range(r_rules)]
    beta_vec = bcast(r_rules)

    def compute(j, _):
        acc = 1.0 - beta_vec
        rows = j * _L + lanes
        for r in range(r_rules):
            cols = riota[pl.ds(pl.multiple_of(r * _L, _L), _L)]
            m = plsc.load_gather(matv2, [rows, cols])
            off = pl.multiple_of((r * n_groups + j) * _L, _L)
            z = pv[pl.ds(off, _L)]
            act = jnp.where(m == m, 1.0 / (1.0 + jnp.exp(-z)), 0.0)
            acc = acc + w_vecs[r] * act
        off = pl.multiple_of(j * _L, _L)
        resv[pl.ds(off, _L)] = jnp.minimum(jnp.maximum(acc, 0.0), 1.0)
        return 0

    lax.fori_loop(0, n_groups, compute, 0)

    pltpu.sync_copy(resv, out_hbm.at[pl.ds(base, bpw)])

    @pl.when(wid == 0)
    def _():
        is_w = lanes < r_rules
        s1 = jnp.sum(jnp.where(is_w, jnp.maximum(wv_all - 1.0, 0.0), 0.0))
        sum_w = jnp.sum(jnp.where(is_w, wv_all, 0.0))
        beta_s = jnp.sum(jnp.where(lanes == r_rules, wv_all, 0.0))
        slack = s1 + jnp.maximum(beta_s - sum_w, 0.0)
        slkv[...] = jnp.zeros((_L,), jnp.float32) + slack
        pltpu.sync_copy(slkv, slk_hbm)


@functools.partial(jax.jit, static_argnums=(4, 5, 6))
def _sc_call(x, mat, rp, wb, b, n_rows, r_rules):
    info = plsc.get_sparse_core_info()
    nc, ns = info.num_cores, info.num_subcores
    nw = nc * ns
    assert b % (nw * _CHUNK) == 0
    bpw = b // nw

    body = functools.partial(_sc_body, nc, ns, bpw, n_rows, r_rules)
    return pl.kernel(
        body,
        out_type=[
            jax.ShapeDtypeStruct((b,), jnp.float32),
            jax.ShapeDtypeStruct((_L,), jnp.float32),
        ],
        mesh=plsc.VectorSubcoreMesh(core_axis_name="c", subcore_axis_name="s"),
        compiler_params=pltpu.CompilerParams(
            needs_layout_passes=False, use_tc_tiling_on_sc=False),
        scratch_types=[
            pltpu.VMEM((bpw,), jnp.int32),
            pltpu.VMEM((bpw, r_rules), jnp.float32),
            pltpu.VMEM((bpw * r_rules // _CHUNK, _CHUNK), jnp.int32),
            pltpu.VMEM((bpw * r_rules,), jnp.float32),
            pltpu.VMEM((bpw,), jnp.float32),
            pltpu.VMEM((_L,), jnp.float32),
            pltpu.VMEM((r_rules * _L,), jnp.int32),
            pltpu.VMEM((_L,), jnp.float32),
            pltpu.SemaphoreType.DMA,
            pltpu.SemaphoreType.DMA,
        ],
    )(x, mat, rp, wb)


def kernel(x, mat, rule_params, w, beta):
    b = x.shape[0]
    r_rules = mat.shape[1]
    n_rows = rule_params.shape[1]
    x = x.astype(jnp.int32)
    wb = jnp.concatenate([
        w.astype(jnp.float32),
        jnp.reshape(beta, (1,)).astype(jnp.float32),
        jnp.zeros((_L - r_rules - 1,), jnp.float32),
    ])
    out, slk = _sc_call(x, mat, rule_params, wb, b, n_rows, r_rules)
    return out.reshape(-1, 1), slk[0]
